# baseline (speedup 1.0000x reference)
import dataclasses
import functools

import jax
import jax.numpy as jnp
from jax import lax
from jax.experimental import pallas as pl
from jax.experimental.pallas import tpu as pltpu
from jax.experimental.pallas import tpu_sc as plsc

N = 10000
D = 128
NPAD = 10240
NSUB = 16
RPT = NPAD // NSUB
WIN = 128
WPT = 160
NWIN = NSUB * WPT
EPAD = NWIN * WIN
CHW = 16


def _pre_body(x_ref, w_ref, o_ref):
    y = lax.dot_general(x_ref[...], w_ref[...], (((1,), (1,)), ((), ())),
                        preferred_element_type=jnp.float32)
    o_ref[...] = jnp.maximum(y, 0.0)


def _pre(x, w):
    br = 2000
    return pl.pallas_call(
        _pre_body,
        grid=(N // br,),
        in_specs=[pl.BlockSpec((br, D), lambda i: (i, 0)),
                  pl.BlockSpec((D, D), lambda i: (0, 0))],
        out_specs=pl.BlockSpec((br, D), lambda i: (i, 0)),
        out_shape=jax.ShapeDtypeStruct((N, D), jnp.float32),
    )(x, w)


def _post_body(x_ref, s_ref, c_ref, w1_ref, w2_ref, b_ref, g_ref, bb_ref,
               o_ref):
    x = x_ref[...]
    cnt = lax.dot_general(c_ref[...], jnp.ones((NSUB, D), jnp.float32),
                          (((0,), (0,)), ((), ())),
                          preferred_element_type=jnp.float32)
    cnt = jnp.maximum(cnt, 1.0)
    agg = s_ref[...] / cnt
    h = (lax.dot_general(x, w1_ref[...], (((1,), (1,)), ((), ())),
                         preferred_element_type=jnp.float32)
         + lax.dot_general(agg, w2_ref[...], (((1,), (1,)), ((), ())),
                           preferred_element_type=jnp.float32)
         + b_ref[...])
    h = jnp.maximum(h, 0.0) + x
    mu = jnp.mean(h, axis=-1, keepdims=True)
    var = jnp.mean((h - mu) ** 2, axis=-1, keepdims=True)
    o_ref[...] = (h - mu) / jnp.sqrt(var + 1e-5) * g_ref[...] + bb_ref[...]


def _post(x, s_pad, c_pad, ww, wb, g, b):
    br = 2048
    w1 = ww[:, :D]
    w2 = ww[:, D:]
    return pl.pallas_call(
        _post_body,
        grid=(pl.cdiv(N, br),),
        in_specs=[
            pl.BlockSpec((br, D), lambda i: (i, 0)),
            pl.BlockSpec((br, D), lambda i: (i, 0)),
            pl.BlockSpec((NSUB, br), lambda i: (0, i)),
            pl.BlockSpec((D, D), lambda i: (0, 0)),
            pl.BlockSpec((D, D), lambda i: (0, 0)),
            pl.BlockSpec((1, D), lambda i: (0, 0)),
            pl.BlockSpec((1, D), lambda i: (0, 0)),
            pl.BlockSpec((1, D), lambda i: (0, 0)),
        ],
        out_specs=pl.BlockSpec((br, D), lambda i: (i, 0)),
        out_shape=jax.ShapeDtypeStruct((N, D), jnp.float32),
    )(x, s_pad, c_pad, w1, w2, wb.reshape(1, D), g.reshape(1, D),
      b.reshape(1, D))


_MESH = plsc.VectorSubcoreMesh(core_axis_name="c", subcore_axis_name="s")


def _sc_counts(idx_u, idx_i):
    f32 = jnp.float32
    out_type = [jax.ShapeDtypeStruct((NSUB, NPAD), f32),
                jax.ShapeDtypeStruct((NSUB, NPAD), f32)]
    scratch = [pltpu.VMEM((CHW, 2, WIN), jnp.int32),
               pltpu.VMEM((NPAD,), f32)]
    cp = pltpu.CompilerParams()
    if "needs_layout_passes" in pltpu.CompilerParams.__dataclass_fields__:
        cp = dataclasses.replace(cp, needs_layout_passes=False)

    @functools.partial(pl.kernel, mesh=_MESH, out_type=out_type,
                       scratch_types=scratch, compiler_params=cp)
    def k(iu_h, ii_h, oci_h, ocu_h, ibuf, hist):
        cid = lax.axis_index("c")
        sid = lax.axis_index("s")

        @pl.loop(0, NPAD, step=16)
        def _(i):
            hist[pl.ds(i, 16)] = jnp.zeros((16,), f32)

        def run(idx_h, oc_h):
            ones = jnp.ones((16,), f32)

            @pl.loop(0, WPT, step=CHW)
            def _(g):
                pltpu.sync_copy(idx_h.at[pl.ds(sid * WPT + g, CHW)], ibuf)
                for j in range(CHW):
                    for kk in range(WIN // 16):
                        iv = ibuf[j, 1, pl.ds(16 * kk, 16)]
                        plsc.addupdate_scatter(hist, [iv], ones)

            pltpu.sync_copy(hist, oc_h.at[sid])

        @pl.when(cid == 0)
        def _():
            run(iu_h, oci_h)

        @pl.when(cid == 1)
        def _():
            run(ii_h, ocu_h)

    return k(idx_u, idx_i)


def _sc_sums(yu32, yi32, idx_u, idx_i, zf):
    f32 = jnp.float32
    out_type = [jax.ShapeDtypeStruct((NPAD, D), f32),
                jax.ShapeDtypeStruct((NPAD, D), f32)]
    scratch = [pltpu.VMEM((CHW, 2, WIN), jnp.int32),
               pltpu.VMEM((WIN, D // 2), jnp.int32),
               pltpu.VMEM((WIN, D // 2), jnp.int32),
               pltpu.VMEM((WIN, D), f32),
               pltpu.VMEM_SHARED((NPAD, D), f32),
               pltpu.SemaphoreType.DMA,
               pltpu.SemaphoreType.DMA,
               pltpu.SemaphoreType.DMA]
    cp = pltpu.CompilerParams()
    if "needs_layout_passes" in pltpu.CompilerParams.__dataclass_fields__:
        cp = dataclasses.replace(cp, needs_layout_passes=False,
                                 use_tc_tiling_on_sc=False)

    @functools.partial(pl.kernel, mesh=_MESH, out_type=out_type,
                       scratch_types=scratch, compiler_params=cp)
    def k(yu_h, yi_h, iu_h, ii_h, zf_h, osi_h, osu_h,
          ibuf, r0, r1, fbuf, accf, g0, g1, csem):
        rows = [r0, r1]
        gsem = [g0, g1]
        cid = lax.axis_index("c")
        sid = lax.axis_index("s")

        def run(y_h, idx_h, os_h):
            pltpu.sync_copy(zf_h.at[pl.ds(sid * RPT, RPT)],
                            accf.at[pl.ds(sid * RPT, RPT)])
            plsc.subcore_barrier()

            @pl.loop(0, WPT, step=CHW)
            def _(g):
                pltpu.sync_copy(idx_h.at[pl.ds(sid * WPT + g, CHW)], ibuf)
                pltpu.async_copy(y_h.at[ibuf.at[0, 0]], rows[0], gsem[0])
                for j in range(CHW):
                    p = j % 2
                    q = 1 - p
                    if j + 1 < CHW:
                        pltpu.async_copy(y_h.at[ibuf.at[j + 1, 0]], rows[q],
                                         gsem[q])
                    pltpu.make_async_copy(y_h.at[ibuf.at[0, 0]], rows[p],
                                          gsem[p]).wait()
                    if j >= 1:
                        pltpu.make_async_copy(fbuf, accf.at[ibuf.at[0, 1]],
                                              csem).wait()
                    rb = rows[p]

                    @pl.loop(0, WIN, step=4)
                    def _(r0):
                        for rr in range(4):
                            r = r0 + rr
                            for m in range(D // 32):
                                w = rb[r, pl.ds(16 * m, 16)]
                                fbuf[r, pl.ds(16 * m, 16)] = plsc.bitcast(
                                    w << 16, f32)
                                fbuf[r, pl.ds(D // 2 + 16 * m, 16)] = \
                                    plsc.bitcast(w & jnp.int32(-65536), f32)

                    pltpu.async_copy(fbuf, accf.at[ibuf.at[j, 1]], csem,
                                     add=True)
                pltpu.make_async_copy(fbuf, accf.at[ibuf.at[0, 1]],
                                      csem).wait()

            plsc.subcore_barrier()
            pltpu.sync_copy(accf.at[pl.ds(sid * RPT, RPT)],
                            os_h.at[pl.ds(sid * RPT, RPT)])

        @pl.when(cid == 0)
        def _():
            run(yu_h, iu_h, osi_h)

        @pl.when(cid == 1)
        def _():
            run(yi_h, ii_h, osu_h)

    return k(yu32, yi32, idx_u, idx_i, zf)


def _pack_bf16(y):
    yb = y.astype(jnp.bfloat16)
    yp = yb.reshape(y.shape[0], 2, D // 2).transpose(0, 2, 1)
    return jax.lax.bitcast_convert_type(yp, jnp.int32)


def _prep_edges(edge_index):
    src = edge_index[0]
    dst = edge_index[1]
    pad = EPAD - src.shape[0]
    src_p = jnp.concatenate([src, jnp.zeros((pad,), jnp.int32)])
    dst_p = jnp.concatenate([dst, jnp.full((pad,), N, jnp.int32)])
    return jnp.stack([src_p.reshape(NWIN, WIN),
                      dst_p.reshape(NWIN, WIN)], axis=1)


def kernel(x_user, x_item, edge_index_user_clicks_item,
           edge_index_item_rev_clicks_user, Wagg_user_clicks,
           Wagg_item_rev_clicks, W_user_w, W_user_b, W_item_w, W_item_b,
           ln_user_g, ln_user_b, ln_item_g, ln_item_b):
    idx_u = _prep_edges(edge_index_user_clicks_item)
    idx_i = _prep_edges(edge_index_item_rev_clicks_user)
    zf = jnp.zeros((NPAD, D), jnp.float32)
    cnt_i, cnt_u = _sc_counts(idx_u, idx_i)
    yu = _pre(x_user, Wagg_user_clicks)
    yi = _pre(x_item, Wagg_item_rev_clicks)
    sum_i, sum_u = _sc_sums(_pack_bf16(yu), _pack_bf16(yi), idx_u, idx_i,
                            zf)
    out_user = _post(x_user, sum_u, cnt_u, W_user_w, W_user_b,
                     ln_user_g, ln_user_b)
    out_item = _post(x_item, sum_i, cnt_i, W_item_w, W_item_b,
                     ln_item_g, ln_item_b)
    return (out_user, out_item)


# speedup vs baseline: 1.3283x; 1.3283x over previous
import dataclasses
import functools

import jax
import jax.numpy as jnp
from jax import lax
from jax.experimental import pallas as pl
from jax.experimental.pallas import tpu as pltpu
from jax.experimental.pallas import tpu_sc as plsc

N = 10000
D = 128
NPAD = 10240
NSUB = 16
RPT = NPAD // NSUB
WIN = 128
WPT = 160
NWIN = NSUB * WPT
EPAD = NWIN * WIN
CHW = 16


def _pre_body(x_ref, w_ref, o_ref):
    y = lax.dot_general(x_ref[...], w_ref[...], (((1,), (1,)), ((), ())),
                        preferred_element_type=jnp.float32)
    o_ref[...] = jnp.maximum(y, 0.0)


def _pre(x, w):
    br = 2000
    return pl.pallas_call(
        _pre_body,
        grid=(N // br,),
        in_specs=[pl.BlockSpec((br, D), lambda i: (i, 0)),
                  pl.BlockSpec((D, D), lambda i: (0, 0))],
        out_specs=pl.BlockSpec((br, D), lambda i: (i, 0)),
        out_shape=jax.ShapeDtypeStruct((N, D), jnp.float32),
    )(x, w)


def _post_body(x_ref, s_ref, c_ref, w1_ref, w2_ref, b_ref, g_ref, bb_ref,
               o_ref):
    x = x_ref[...]
    cnt = lax.dot_general(c_ref[...], jnp.ones((NSUB, D), jnp.float32),
                          (((0,), (0,)), ((), ())),
                          preferred_element_type=jnp.float32)
    cnt = jnp.maximum(cnt, 1.0)
    agg = s_ref[...] / cnt
    h = (lax.dot_general(x, w1_ref[...], (((1,), (1,)), ((), ())),
                         preferred_element_type=jnp.float32)
         + lax.dot_general(agg, w2_ref[...], (((1,), (1,)), ((), ())),
                           preferred_element_type=jnp.float32)
         + b_ref[...])
    h = jnp.maximum(h, 0.0) + x
    mu = jnp.mean(h, axis=-1, keepdims=True)
    var = jnp.mean((h - mu) ** 2, axis=-1, keepdims=True)
    o_ref[...] = (h - mu) / jnp.sqrt(var + 1e-5) * g_ref[...] + bb_ref[...]


def _post(x, s_pad, c_pad, ww, wb, g, b):
    br = 2048
    w1 = ww[:, :D]
    w2 = ww[:, D:]
    return pl.pallas_call(
        _post_body,
        grid=(pl.cdiv(N, br),),
        in_specs=[
            pl.BlockSpec((br, D), lambda i: (i, 0)),
            pl.BlockSpec((br, D), lambda i: (i, 0)),
            pl.BlockSpec((NSUB, br), lambda i: (0, i)),
            pl.BlockSpec((D, D), lambda i: (0, 0)),
            pl.BlockSpec((D, D), lambda i: (0, 0)),
            pl.BlockSpec((1, D), lambda i: (0, 0)),
            pl.BlockSpec((1, D), lambda i: (0, 0)),
            pl.BlockSpec((1, D), lambda i: (0, 0)),
        ],
        out_specs=pl.BlockSpec((br, D), lambda i: (i, 0)),
        out_shape=jax.ShapeDtypeStruct((N, D), jnp.float32),
    )(x, s_pad, c_pad, w1, w2, wb.reshape(1, D), g.reshape(1, D),
      b.reshape(1, D))


_MESH = plsc.VectorSubcoreMesh(core_axis_name="c", subcore_axis_name="s")


def _sc_counts(idx_u, idx_i):
    f32 = jnp.float32
    out_type = [jax.ShapeDtypeStruct((NSUB, NPAD), f32),
                jax.ShapeDtypeStruct((NSUB, NPAD), f32)]
    scratch = [pltpu.VMEM((CHW, 2, WIN), jnp.int32),
               pltpu.VMEM((NPAD,), f32)]
    cp = pltpu.CompilerParams()
    if "needs_layout_passes" in pltpu.CompilerParams.__dataclass_fields__:
        cp = dataclasses.replace(cp, needs_layout_passes=False)

    @functools.partial(pl.kernel, mesh=_MESH, out_type=out_type,
                       scratch_types=scratch, compiler_params=cp)
    def k(iu_h, ii_h, oci_h, ocu_h, ibuf, hist):
        cid = lax.axis_index("c")
        sid = lax.axis_index("s")

        @pl.loop(0, NPAD, step=16)
        def _(i):
            hist[pl.ds(i, 16)] = jnp.zeros((16,), f32)

        def run(idx_h, oc_h):
            ones = jnp.ones((16,), f32)

            @pl.loop(0, WPT, step=CHW)
            def _(g):
                pltpu.sync_copy(idx_h.at[pl.ds(sid * WPT + g, CHW)], ibuf)
                for j in range(CHW):
                    for kk in range(WIN // 16):
                        iv = ibuf[j, 1, pl.ds(16 * kk, 16)]
                        plsc.addupdate_scatter(hist, [iv], ones)

            pltpu.sync_copy(hist, oc_h.at[sid])

        @pl.when(cid == 0)
        def _():
            run(iu_h, oci_h)

        @pl.when(cid == 1)
        def _():
            run(ii_h, ocu_h)

    return k(idx_u, idx_i)


def _sc_sums(yu32, yi32, idx_u, idx_i, zf):
    f32 = jnp.float32
    out_type = [jax.ShapeDtypeStruct((NPAD, D), f32),
                jax.ShapeDtypeStruct((NPAD, D), f32)]
    scratch = [pltpu.VMEM((CHW, 2, WIN), jnp.int32),
               pltpu.VMEM((WIN, D // 2), jnp.int32),
               pltpu.VMEM((WIN, D // 2), jnp.int32),
               pltpu.VMEM((WIN, D), f32),
               pltpu.VMEM_SHARED((NPAD, D), f32),
               pltpu.SemaphoreType.DMA,
               pltpu.SemaphoreType.DMA,
               pltpu.SemaphoreType.DMA]
    cp = pltpu.CompilerParams()
    if "needs_layout_passes" in pltpu.CompilerParams.__dataclass_fields__:
        cp = dataclasses.replace(cp, needs_layout_passes=False,
                                 use_tc_tiling_on_sc=False)

    @functools.partial(pl.kernel, mesh=_MESH, out_type=out_type,
                       scratch_types=scratch, compiler_params=cp)
    def k(yu_h, yi_h, iu_h, ii_h, zf_h, osi_h, osu_h,
          ibuf, r0, r1, fbuf, accf, g0, g1, csem):
        rows = [r0, r1]
        gsem = [g0, g1]
        cid = lax.axis_index("c")
        sid = lax.axis_index("s")

        def run(y_h, idx_h, os_h):
            pltpu.sync_copy(zf_h.at[pl.ds(sid * RPT, RPT)],
                            accf.at[pl.ds(sid * RPT, RPT)])
            plsc.subcore_barrier()

            @pl.loop(0, WPT, step=CHW)
            def _(g):
                pltpu.sync_copy(idx_h.at[pl.ds(sid * WPT + g, CHW)], ibuf)
                pltpu.async_copy(y_h.at[ibuf.at[0, 0]], rows[0], gsem[0])
                for j in range(CHW):
                    p = j % 2
                    q = 1 - p
                    if j + 1 < CHW:
                        pltpu.async_copy(y_h.at[ibuf.at[j + 1, 0]], rows[q],
                                         gsem[q])
                    pltpu.make_async_copy(y_h.at[ibuf.at[0, 0]], rows[p],
                                          gsem[p]).wait()
                    if j >= 1:
                        pltpu.make_async_copy(fbuf, accf.at[ibuf.at[0, 1]],
                                              csem).wait()
                    rb = rows[p]

                    @plsc.parallel_loop(0, WIN, unroll=4)
                    def _(r):
                        for m in range(D // 32):
                            w = rb[r, pl.ds(16 * m, 16)]
                            fbuf[r, pl.ds(16 * m, 16)] = plsc.bitcast(
                                w << 16, f32)
                            fbuf[r, pl.ds(D // 2 + 16 * m, 16)] = \
                                plsc.bitcast(w & jnp.int32(-65536), f32)

                    pltpu.async_copy(fbuf, accf.at[ibuf.at[j, 1]], csem,
                                     add=True)
                pltpu.make_async_copy(fbuf, accf.at[ibuf.at[0, 1]],
                                      csem).wait()

            plsc.subcore_barrier()
            pltpu.sync_copy(accf.at[pl.ds(sid * RPT, RPT)],
                            os_h.at[pl.ds(sid * RPT, RPT)])

        @pl.when(cid == 0)
        def _():
            run(yu_h, iu_h, osi_h)

        @pl.when(cid == 1)
        def _():
            run(yi_h, ii_h, osu_h)

    return k(yu32, yi32, idx_u, idx_i, zf)


def _pack_bf16(y):
    yb = y.astype(jnp.bfloat16)
    yp = yb.reshape(y.shape[0], 2, D // 2).transpose(0, 2, 1)
    return jax.lax.bitcast_convert_type(yp, jnp.int32)


def _prep_edges(edge_index):
    src = edge_index[0]
    dst = edge_index[1]
    pad = EPAD - src.shape[0]
    src_p = jnp.concatenate([src, jnp.zeros((pad,), jnp.int32)])
    dst_p = jnp.concatenate([dst, jnp.full((pad,), N, jnp.int32)])
    return jnp.stack([src_p.reshape(NWIN, WIN),
                      dst_p.reshape(NWIN, WIN)], axis=1)


def kernel(x_user, x_item, edge_index_user_clicks_item,
           edge_index_item_rev_clicks_user, Wagg_user_clicks,
           Wagg_item_rev_clicks, W_user_w, W_user_b, W_item_w, W_item_b,
           ln_user_g, ln_user_b, ln_item_g, ln_item_b):
    idx_u = _prep_edges(edge_index_user_clicks_item)
    idx_i = _prep_edges(edge_index_item_rev_clicks_user)
    zf = jnp.zeros((NPAD, D), jnp.float32)
    cnt_i, cnt_u = _sc_counts(idx_u, idx_i)
    yu = _pre(x_user, Wagg_user_clicks)
    yi = _pre(x_item, Wagg_item_rev_clicks)
    sum_i, sum_u = _sc_sums(_pack_bf16(yu), _pack_bf16(yi), idx_u, idx_i,
                            zf)
    out_user = _post(x_user, sum_u, cnt_u, W_user_w, W_user_b,
                     ln_user_g, ln_user_b)
    out_item = _post(x_item, sum_i, cnt_i, W_item_w, W_item_b,
                     ln_item_g, ln_item_b)
    return (out_user, out_item)
